# Initial kernel scaffold; baseline (speedup 1.0000x reference)
import jax
import jax.numpy as jnp
from jax.experimental import pallas as pl


def kernel(tgt_seq, scale, pos_h_embedding, pos_w_embedding):
    raise NotImplementedError("write your pallas kernel here")


# speedup vs baseline: 9.1966x; 9.1966x over previous
import functools
import math

import jax
import jax.numpy as jnp
from jax import lax
from jax.experimental import pallas as pl
from jax.experimental.pallas import tpu as pltpu
from jax.experimental.pallas import tpu_sc as plsc

HEIGHT = 20
WIDTH = 32
D_HALF = 64
N_SPECIAL = 2

TABLE_ROWS = 648
D_MODEL = 2 * D_HALF
GROUP = 128


def _table_body(scale_ref, h_ref, w_ref, out_ref):
    s = scale_ref[0, 0]
    r = lax.broadcasted_iota(jnp.int32, (TABLE_ROWS, WIDTH), 0)
    c = lax.broadcasted_iota(jnp.int32, (TABLE_ROWS, WIDTH), 1)
    a = jnp.maximum(r - N_SPECIAL, 0).astype(jnp.float32)
    wdiv = jnp.float32(WIDTH) / s.astype(jnp.float32)
    off = (s // 2).astype(jnp.float32)
    q = jnp.floor(a / wdiv)
    xi = (q + off).astype(jnp.int32)
    yi = (a - q * wdiv + off).astype(jnp.int32)
    oh_x = (c == xi).astype(jnp.float32)
    oh_y = (c == yi).astype(jnp.float32)
    pe_x = lax.dot(oh_x, h_ref[:, :], preferred_element_type=jnp.float32)
    pe_y = lax.dot(oh_y, w_ref[:, :], preferred_element_type=jnp.float32)
    valid = (r[:, :1] >= N_SPECIAL).astype(jnp.float32)
    out_ref[:, :] = jnp.concatenate([pe_x, pe_y], axis=1) * valid


def _build_table(scale, pos_h_embedding, pos_w_embedding):
    h_pad = jnp.zeros((WIDTH, D_HALF), jnp.float32).at[:HEIGHT].set(pos_h_embedding)
    scale_arr = jnp.asarray(scale, jnp.int32).reshape(1, 1)
    return pl.pallas_call(
        _table_body,
        out_shape=jax.ShapeDtypeStruct((TABLE_ROWS, D_MODEL), jnp.float32),
        in_specs=[
            pl.BlockSpec(memory_space=pltpu.SMEM),
            pl.BlockSpec(memory_space=pltpu.VMEM),
            pl.BlockSpec(memory_space=pltpu.VMEM),
        ],
        out_specs=pl.BlockSpec(memory_space=pltpu.VMEM),
    )(scale_arr, h_pad, pos_w_embedding)


def _sc_gather(table, idx3d):
    nw, rpw, _ = idx3d.shape

    mesh = plsc.VectorSubcoreMesh(core_axis_name="c", subcore_axis_name="s")

    @functools.partial(
        pl.kernel,
        mesh=mesh,
        out_type=jax.ShapeDtypeStruct((nw, rpw, GROUP, D_MODEL), jnp.float32),
        scratch_types=[
            pltpu.VMEM((rpw, GROUP), jnp.int32),
            pltpu.VMEM((GROUP, D_MODEL), jnp.float32),
            pltpu.SemaphoreType.DMA,
        ],
    )
    def k(table_hbm, idx_hbm, out_hbm, idx_v, rows_v, sem):
        ncores = jax.lax.axis_size("c")
        wid = lax.axis_index("s") * ncores + lax.axis_index("c")
        pltpu.sync_copy(idx_hbm.at[wid], idx_v)

        def body(j, carry):
            pltpu.async_copy(table_hbm.at[idx_v.at[j]], rows_v, sem).wait()
            pltpu.sync_copy(rows_v, out_hbm.at[wid, j])
            return carry

        lax.fori_loop(0, rpw, body, 0)

    return k(table, idx3d)


def kernel(tgt_seq, scale, pos_h_embedding, pos_w_embedding):
    b, t = tgt_seq.shape
    table = _build_table(scale, pos_h_embedding, pos_w_embedding)
    info = plsc.get_sparse_core_info()
    nw = info.num_cores * info.num_subcores
    idx3d = tgt_seq.reshape(nw, b * t // (nw * GROUP), GROUP)
    out = _sc_gather(table, idx3d)
    return out.reshape(b, t, D_MODEL)


# speedup vs baseline: 9.7918x; 1.0647x over previous
import functools
import math

import jax
import jax.numpy as jnp
from jax import lax
from jax.experimental import pallas as pl
from jax.experimental.pallas import tpu as pltpu
from jax.experimental.pallas import tpu_sc as plsc

HEIGHT = 20
WIDTH = 32
D_HALF = 64
N_SPECIAL = 2

TABLE_ROWS = 648
D_MODEL = 2 * D_HALF
GROUP = 128


def _table_body(scale_ref, h_ref, w_ref, out_ref):
    s = scale_ref[0, 0]
    r = lax.broadcasted_iota(jnp.int32, (TABLE_ROWS, WIDTH), 0)
    c = lax.broadcasted_iota(jnp.int32, (TABLE_ROWS, WIDTH), 1)
    a = jnp.maximum(r - N_SPECIAL, 0).astype(jnp.float32)
    wdiv = jnp.float32(WIDTH) / s.astype(jnp.float32)
    off = (s // 2).astype(jnp.float32)
    q = jnp.floor(a / wdiv)
    xi = (q + off).astype(jnp.int32)
    yi = (a - q * wdiv + off).astype(jnp.int32)
    oh_x = (c == xi).astype(jnp.float32)
    oh_y = (c == yi).astype(jnp.float32)
    pe_x = lax.dot(oh_x, h_ref[:, :], preferred_element_type=jnp.float32)
    pe_y = lax.dot(oh_y, w_ref[:, :], preferred_element_type=jnp.float32)
    valid = (r[:, :1] >= N_SPECIAL).astype(jnp.float32)
    out_ref[:, :] = jnp.concatenate([pe_x, pe_y], axis=1) * valid


def _build_table(scale, pos_h_embedding, pos_w_embedding):
    h_pad = jnp.zeros((WIDTH, D_HALF), jnp.float32).at[:HEIGHT].set(pos_h_embedding)
    scale_arr = jnp.asarray(scale, jnp.int32).reshape(1, 1)
    return pl.pallas_call(
        _table_body,
        out_shape=jax.ShapeDtypeStruct((TABLE_ROWS, D_MODEL), jnp.float32),
        in_specs=[
            pl.BlockSpec(memory_space=pltpu.SMEM),
            pl.BlockSpec(memory_space=pltpu.VMEM),
            pl.BlockSpec(memory_space=pltpu.VMEM),
        ],
        out_specs=pl.BlockSpec(memory_space=pltpu.VMEM),
    )(scale_arr, h_pad, pos_w_embedding)


def _sc_gather(table, idx3d):
    nw, rpw, _ = idx3d.shape

    mesh = plsc.VectorSubcoreMesh(core_axis_name="c", subcore_axis_name="s")

    @functools.partial(
        pl.kernel,
        mesh=mesh,
        out_type=jax.ShapeDtypeStruct((nw, rpw, GROUP, D_MODEL), jnp.float32),
        scratch_types=[
            pltpu.VMEM((rpw, GROUP), jnp.int32),
            pltpu.VMEM((GROUP, D_MODEL), jnp.float32),
            pltpu.VMEM((GROUP, D_MODEL), jnp.float32),
            pltpu.SemaphoreType.DMA,
            pltpu.SemaphoreType.DMA,
        ],
    )
    def k(table_hbm, idx_hbm, out_hbm, idx_v, buf_a, buf_b, sem_a, sem_b):
        ncores = jax.lax.axis_size("c")
        wid = lax.axis_index("s") * ncores + lax.axis_index("c")
        pltpu.sync_copy(idx_hbm.at[wid], idx_v)

        def gather(j, buf, sem):
            return pltpu.make_async_copy(table_hbm.at[idx_v.at[j]], buf, sem)

        gather(0, buf_a, sem_a).start()

        def body(g, carry):
            j0 = 2 * g
            j1 = j0 + 1
            gather(j1, buf_b, sem_b).start()
            gather(j0, buf_a, sem_a).wait()
            pltpu.sync_copy(buf_a, out_hbm.at[wid, j0])

            @pl.when(j1 + 1 < rpw)
            def _():
                gather(j1 + 1, buf_a, sem_a).start()

            gather(j1, buf_b, sem_b).wait()
            pltpu.sync_copy(buf_b, out_hbm.at[wid, j1])
            return carry

        lax.fori_loop(0, rpw // 2, body, 0)

    return k(table, idx3d)


def kernel(tgt_seq, scale, pos_h_embedding, pos_w_embedding):
    b, t = tgt_seq.shape
    table = _build_table(scale, pos_h_embedding, pos_w_embedding)
    info = plsc.get_sparse_core_info()
    nw = info.num_cores * info.num_subcores
    idx3d = tgt_seq.reshape(nw, b * t // (nw * GROUP), GROUP)
    out = _sc_gather(table, idx3d)
    return out.reshape(b, t, D_MODEL)
